# Initial kernel scaffold; baseline (speedup 1.0000x reference)
import jax
import jax.numpy as jnp
from jax.experimental import pallas as pl


def kernel(x, edge_index, W1, b1, W2, b2):
    raise NotImplementedError("write your pallas kernel here")


# speedup vs baseline: 3.6106x; 3.6106x over previous
import functools

import jax
import jax.numpy as jnp
from jax import lax
from jax.experimental import pallas as pl
from jax.experimental.pallas import tpu as pltpu
from jax.experimental.pallas import tpu_sc as plsc

N = 10000
E = 320000
D = 128
HH = 64
NP = 10240
NTILES = 16
ROWS_PER_TILE = NP // NTILES
CH = 128
NCHUNK = 157
EDGES_PER_TILE = NCHUNK * CH
E_PAD = EDGES_PER_TILE * NTILES

_SELU_ALPHA = 1.6732632423543772
_SELU_SCALE = 1.0507009873554805


@functools.partial(
    pl.kernel,
    mesh=plsc.VectorSubcoreMesh(core_axis_name="c", subcore_axis_name="s"),
    out_type=(
        jax.ShapeDtypeStruct((NP, HH), jnp.float32),
        jax.ShapeDtypeStruct((NP, HH), jnp.float32),
    ),
    scratch_types=[
        pltpu.VMEM_SHARED((NP, HH), jnp.float32),
        pltpu.VMEM((CH,), jnp.int32),
        pltpu.VMEM((CH,), jnp.int32),
        pltpu.VMEM((CH, HH), jnp.float32),
        pltpu.SemaphoreType.DMA,
    ],
    compiler_params=pltpu.CompilerParams(use_tc_tiling_on_sc=False),
)
def _agg(ha, hb, src, dst, oa, ob, acc, idx_s, idx_d, rows, sem):
    c = lax.axis_index("c")
    s = lax.axis_index("s")

    def body(table, out):
        r0 = s * ROWS_PER_TILE
        pltpu.sync_copy(table.at[pl.ds(r0, ROWS_PER_TILE)],
                        acc.at[pl.ds(r0, ROWS_PER_TILE)])
        plsc.subcore_barrier()

        e0 = s * EDGES_PER_TILE

        def step(j, carry):
            base = e0 + j * CH
            pltpu.sync_copy(src.at[pl.ds(base, CH)], idx_s)
            pltpu.sync_copy(dst.at[pl.ds(base, CH)], idx_d)
            pltpu.async_copy(table.at[idx_s], rows, sem).wait()
            pltpu.sync_copy(rows, acc.at[idx_d], add=True)
            return carry

        lax.fori_loop(0, NCHUNK, step, 0)
        plsc.subcore_barrier()
        pltpu.sync_copy(acc.at[pl.ds(r0, ROWS_PER_TILE)],
                        out.at[pl.ds(r0, ROWS_PER_TILE)])

    @pl.when(c == 0)
    def _():
        body(ha, oa)

    @pl.when(c == 1)
    def _():
        body(hb, ob)


def _mlp_body(oa_ref, ob_ref, w1a_ref, w1b_ref, b1_ref, pa_ref, pb_ref):
    z = (jnp.dot(oa_ref[...], w1a_ref[...], preferred_element_type=jnp.float32)
         + jnp.dot(ob_ref[...], w1b_ref[...], preferred_element_type=jnp.float32)
         + b1_ref[...])
    act = _SELU_SCALE * jnp.where(z > 0, z, _SELU_ALPHA * (jnp.exp(z) - 1.0))
    pa_ref[...] = act[:, :HH]
    pb_ref[...] = act[:, HH:]


def _mlp(oa, ob, w1a, w1b, b1):
    br = 1024
    grid = (NP // br,)
    return pl.pallas_call(
        _mlp_body,
        grid=grid,
        in_specs=[
            pl.BlockSpec((br, HH), lambda i: (i, 0)),
            pl.BlockSpec((br, HH), lambda i: (i, 0)),
            pl.BlockSpec((HH, D), lambda i: (0, 0)),
            pl.BlockSpec((HH, D), lambda i: (0, 0)),
            pl.BlockSpec((1, D), lambda i: (0, 0)),
        ],
        out_specs=[
            pl.BlockSpec((br, HH), lambda i: (i, 0)),
            pl.BlockSpec((br, HH), lambda i: (i, 0)),
        ],
        out_shape=[
            jax.ShapeDtypeStruct((NP, HH), jnp.float32),
            jax.ShapeDtypeStruct((NP, HH), jnp.float32),
        ],
    )(oa, ob, w1a, w1b, b1)


def _out_body(qa_ref, qb_ref, w2a_ref, w2b_ref, b2_ref, o_ref):
    z = (jnp.dot(qa_ref[...], w2a_ref[...], preferred_element_type=jnp.float32)
         + jnp.dot(qb_ref[...], w2b_ref[...], preferred_element_type=jnp.float32)
         + b2_ref[...])
    m = jnp.max(z, axis=1, keepdims=True)
    lse = jnp.log(jnp.sum(jnp.exp(z - m), axis=1, keepdims=True)) + m
    o_ref[...] = z - lse


def _outk(qa, qb, w2a, w2b, b2):
    br = 1000
    grid = (N // br,)
    return pl.pallas_call(
        _out_body,
        grid=grid,
        in_specs=[
            pl.BlockSpec((br, HH), lambda i: (i, 0)),
            pl.BlockSpec((br, HH), lambda i: (i, 0)),
            pl.BlockSpec((HH, D), lambda i: (0, 0)),
            pl.BlockSpec((HH, D), lambda i: (0, 0)),
            pl.BlockSpec((1, D), lambda i: (0, 0)),
        ],
        out_specs=pl.BlockSpec((br, D), lambda i: (i, 0)),
        out_shape=jax.ShapeDtypeStruct((N, D), jnp.float32),
    )(qa, qb, w2a, w2b, b2)


def kernel(x, edge_index, W1, b1, W2, b2):
    src = edge_index[0]
    dst = edge_index[1]
    pad = E_PAD - E
    src_p = jnp.concatenate([src, jnp.zeros((pad,), jnp.int32)])
    dst_p = jnp.concatenate([dst, jnp.full((pad,), N, jnp.int32)])
    xa = jnp.pad(x[:, :HH], ((0, NP - N), (0, 0)))
    xb = jnp.pad(x[:, HH:], ((0, NP - N), (0, 0)))

    h1a, h1b = _agg(xa, xb, src_p, dst_p)
    h2a, h2b = _agg(h1a, h1b, src_p, dst_p)
    h3a, h3b = _mlp(h2a, h2b, W1[:HH], W1[HH:], b1.reshape(1, D))
    h4a, h4b = _agg(h3a, h3b, src_p, dst_p)
    return _outk(h4a, h4b, W2[:HH], W2[HH:], b2.reshape(1, D))


# speedup vs baseline: 4.7328x; 1.3108x over previous
import functools

import jax
import jax.numpy as jnp
from jax import lax
from jax.experimental import pallas as pl
from jax.experimental.pallas import tpu as pltpu
from jax.experimental.pallas import tpu_sc as plsc

N = 10000
E = 320000
D = 128
HH = 64
NP = 10240
NTILES = 16
ROWS_PER_TILE = NP // NTILES
CH = 128
NCHUNK = 160
EDGES_PER_TILE = NCHUNK * CH
E_PAD = EDGES_PER_TILE * NTILES
NBUF = 5
AHEAD = 3

_SELU_ALPHA = 1.6732632423543772
_SELU_SCALE = 1.0507009873554805


@functools.partial(
    pl.kernel,
    mesh=plsc.VectorSubcoreMesh(core_axis_name="c", subcore_axis_name="s"),
    out_type=(
        jax.ShapeDtypeStruct((NP, HH), jnp.float32),
        jax.ShapeDtypeStruct((NP, HH), jnp.float32),
    ),
    scratch_types=[
        pltpu.VMEM_SHARED((NP, HH), jnp.float32),
        pltpu.VMEM((NCHUNK, CH), jnp.int32),
        pltpu.VMEM((NCHUNK, CH), jnp.int32),
        pltpu.VMEM((NBUF, CH, HH), jnp.float32),
        pltpu.SemaphoreType.DMA((NBUF,)),
        pltpu.SemaphoreType.DMA((NBUF,)),
    ],
    compiler_params=pltpu.CompilerParams(use_tc_tiling_on_sc=False),
)
def _agg(ha, hb, src, dst, oa, ob, acc, idx_s, idx_d, rows, g_sem, s_sem):
    c = lax.axis_index("c")
    s = lax.axis_index("s")

    def body(table, out):
        r0 = s * ROWS_PER_TILE
        pltpu.sync_copy(table.at[pl.ds(r0, ROWS_PER_TILE)],
                        acc.at[pl.ds(r0, ROWS_PER_TILE)])
        c0 = s * NCHUNK
        pltpu.sync_copy(src.at[pl.ds(c0, NCHUNK)], idx_s)
        pltpu.sync_copy(dst.at[pl.ds(c0, NCHUNK)], idx_d)
        plsc.subcore_barrier()

        for b in range(AHEAD):
            pltpu.async_copy(table.at[idx_s.at[b]], rows.at[b], g_sem.at[b])

        def group(g, carry):
            base = g * NBUF
            for b in range(NBUF):
                j = base + b
                pltpu.make_async_copy(table.at[idx_s.at[j]], rows.at[b],
                                      g_sem.at[b]).wait()
                pltpu.async_copy(rows.at[b], acc.at[idx_d.at[j]],
                                 s_sem.at[b], add=True)
                jp = j + AHEAD
                bp = (b + AHEAD) % NBUF

                @pl.when(jp >= NBUF)
                def _():
                    pltpu.make_async_copy(rows.at[bp], acc.at[idx_d.at[0]],
                                          s_sem.at[bp]).wait()

                @pl.when(jp < NCHUNK)
                def _():
                    pltpu.async_copy(table.at[idx_s.at[jp]], rows.at[bp],
                                     g_sem.at[bp])
            return carry

        lax.fori_loop(0, NCHUNK // NBUF, group, 0)

        for i in range(NBUF - AHEAD):
            b = (NCHUNK - NBUF + AHEAD + i) % NBUF
            pltpu.make_async_copy(rows.at[b], acc.at[idx_d.at[0]],
                                  s_sem.at[b]).wait()
        plsc.subcore_barrier()
        pltpu.sync_copy(acc.at[pl.ds(r0, ROWS_PER_TILE)],
                        out.at[pl.ds(r0, ROWS_PER_TILE)])

    @pl.when(c == 0)
    def _():
        body(ha, oa)

    @pl.when(c == 1)
    def _():
        body(hb, ob)


def _mlp_body(oa_ref, ob_ref, w1a_ref, w1b_ref, b1_ref, pa_ref, pb_ref):
    z = (jnp.dot(oa_ref[...], w1a_ref[...], preferred_element_type=jnp.float32)
         + jnp.dot(ob_ref[...], w1b_ref[...], preferred_element_type=jnp.float32)
         + b1_ref[...])
    act = _SELU_SCALE * jnp.where(z > 0, z, _SELU_ALPHA * (jnp.exp(z) - 1.0))
    pa_ref[...] = act[:, :HH]
    pb_ref[...] = act[:, HH:]


def _mlp(oa, ob, w1a, w1b, b1):
    br = 1024
    grid = (NP // br,)
    return pl.pallas_call(
        _mlp_body,
        grid=grid,
        in_specs=[
            pl.BlockSpec((br, HH), lambda i: (i, 0)),
            pl.BlockSpec((br, HH), lambda i: (i, 0)),
            pl.BlockSpec((HH, D), lambda i: (0, 0)),
            pl.BlockSpec((HH, D), lambda i: (0, 0)),
            pl.BlockSpec((1, D), lambda i: (0, 0)),
        ],
        out_specs=[
            pl.BlockSpec((br, HH), lambda i: (i, 0)),
            pl.BlockSpec((br, HH), lambda i: (i, 0)),
        ],
        out_shape=[
            jax.ShapeDtypeStruct((NP, HH), jnp.float32),
            jax.ShapeDtypeStruct((NP, HH), jnp.float32),
        ],
    )(oa, ob, w1a, w1b, b1)


def _out_body(qa_ref, qb_ref, w2a_ref, w2b_ref, b2_ref, o_ref):
    z = (jnp.dot(qa_ref[...], w2a_ref[...], preferred_element_type=jnp.float32)
         + jnp.dot(qb_ref[...], w2b_ref[...], preferred_element_type=jnp.float32)
         + b2_ref[...])
    m = jnp.max(z, axis=1, keepdims=True)
    lse = jnp.log(jnp.sum(jnp.exp(z - m), axis=1, keepdims=True)) + m
    o_ref[...] = z - lse


def _outk(qa, qb, w2a, w2b, b2):
    br = 1000
    grid = (N // br,)
    return pl.pallas_call(
        _out_body,
        grid=grid,
        in_specs=[
            pl.BlockSpec((br, HH), lambda i: (i, 0)),
            pl.BlockSpec((br, HH), lambda i: (i, 0)),
            pl.BlockSpec((HH, D), lambda i: (0, 0)),
            pl.BlockSpec((HH, D), lambda i: (0, 0)),
            pl.BlockSpec((1, D), lambda i: (0, 0)),
        ],
        out_specs=pl.BlockSpec((br, D), lambda i: (i, 0)),
        out_shape=jax.ShapeDtypeStruct((N, D), jnp.float32),
    )(qa, qb, w2a, w2b, b2)


def kernel(x, edge_index, W1, b1, W2, b2):
    src = edge_index[0]
    dst = edge_index[1]
    pad = E_PAD - E
    src_p = jnp.concatenate([src, jnp.zeros((pad,), jnp.int32)])
    dst_p = jnp.concatenate(
        [dst, N + jnp.arange(pad, dtype=jnp.int32) % (NP - N)])
    src2 = src_p.reshape(E_PAD // CH, CH)
    dst2 = dst_p.reshape(E_PAD // CH, CH)
    xa = jnp.pad(x[:, :HH], ((0, NP - N), (0, 0)))
    xb = jnp.pad(x[:, HH:], ((0, NP - N), (0, 0)))

    h1a, h1b = _agg(xa, xb, src2, dst2)
    h2a, h2b = _agg(h1a, h1b, src2, dst2)
    h3a, h3b = _mlp(h2a, h2b, W1[:HH], W1[HH:], b1.reshape(1, D))
    h4a, h4b = _agg(h3a, h3b, src2, dst2)
    return _outk(h4a, h4b, W2[:HH], W2[HH:], b2.reshape(1, D))
